# baseline (speedup 1.0000x reference)
import functools

import jax
import jax.numpy as jnp
from jax.experimental import pallas as pl
from jax.experimental.pallas import tpu as pltpu

KH = 4
KW = 4
N_TAPS = KH * KW
_INTERPRET = False


def _scatter_body(gh_ref, gw_ref, recv_ref, snd_ref, msk_ref, feat_ref, acc_ref,
                  *, edge_block, node_base, node_span, n_ch):
    g = pl.program_id(0)

    @pl.when(g == 0)
    def _init():
        acc_ref[...] = jnp.zeros_like(acc_ref)

    t_iota = jax.lax.broadcasted_iota(jnp.int32, (N_TAPS, n_ch), 0)

    def body(i, _):
        sub = i // 128
        lane = i % 128
        gh = gh_ref[0, sub, lane]
        gw = gw_ref[0, sub, lane]
        m = msk_ref[0, sub, lane]
        r = recv_ref[0, sub, lane] - node_base
        s = snd_ref[0, sub, lane]
        in_range = jnp.logical_and(r >= 0, r < node_span)
        m = jnp.where(in_range, m, 0.0)
        r = jnp.clip(r, 0, node_span - 1)
        h0f = jnp.floor(gh)
        w0f = jnp.floor(gw)
        fh = gh - h0f
        fw = gw - w0f
        h0 = jnp.clip(h0f, 0.0, KH - 1).astype(jnp.int32)
        h1 = jnp.minimum(h0 + 1, KH - 1)
        w0 = jnp.clip(w0f, 0.0, KW - 1).astype(jnp.int32)
        w1 = jnp.minimum(w0 + 1, KW - 1)
        x = feat_ref[pl.ds(s, 1), :]
        zero = jnp.zeros((), jnp.float32)
        wvec = (jnp.where(t_iota == h0 * KW + w0, (1.0 - fh) * (1.0 - fw), zero)
                + jnp.where(t_iota == h0 * KW + w1, (1.0 - fh) * fw, zero)
                + jnp.where(t_iota == h1 * KW + w0, fh * (1.0 - fw), zero)
                + jnp.where(t_iota == h1 * KW + w1, fh * fw, zero)) * m
        acc_ref[pl.ds(r * N_TAPS, N_TAPS), :] += wvec * x
        return 0

    jax.lax.fori_loop(0, edge_block, body, 0)


def _phase1(gh, gw, recv, snd, msk, features, edge_block=1024, node_split=2):
    n_nodes, n_ch = features.shape
    n_blocks = gh.shape[0]
    node_span = n_nodes // node_split
    grid = (n_blocks,)
    sub = edge_block // 128
    smem_spec = pl.BlockSpec((1, sub, 128), lambda g: (g, 0, 0),
                             memory_space=pltpu.SMEM)
    parts = []
    for h in range(node_split):
        parts.append(pl.pallas_call(
            functools.partial(_scatter_body, edge_block=edge_block,
                              node_base=h * node_span, node_span=node_span,
                              n_ch=n_ch),
            grid=grid,
            in_specs=[smem_spec, smem_spec, smem_spec, smem_spec, smem_spec,
                      pl.BlockSpec((n_nodes, n_ch), lambda g: (0, 0))],
            out_specs=pl.BlockSpec((node_span * N_TAPS, n_ch),
                                   lambda g: (0, 0)),
            out_shape=jax.ShapeDtypeStruct((node_span * N_TAPS, n_ch),
                                           jnp.float32),
            interpret=_INTERPRET,
        )(gh, gw, recv, snd, msk, features))
    return jnp.concatenate(parts, axis=0)


def _matmul_body(acc_ref, w_ref, bias_ref, out_ref):
    out_ref[...] = (
        jnp.dot(acc_ref[...], w_ref[...], preferred_element_type=jnp.float32)
        + bias_ref[...])


def _phase2(acc2d, w2d, bias, block_n=1000):
    n_nodes = acc2d.shape[0]
    k = acc2d.shape[1]
    n_out = w2d.shape[1]
    grid = (n_nodes // block_n,)
    return pl.pallas_call(
        _matmul_body,
        grid=grid,
        in_specs=[pl.BlockSpec((block_n, k), lambda g: (g, 0)),
                  pl.BlockSpec((k, n_out), lambda g: (0, 0)),
                  pl.BlockSpec((1, n_out), lambda g: (0, 0))],
        out_specs=pl.BlockSpec((block_n, n_out), lambda g: (g, 0)),
        out_shape=jax.ShapeDtypeStruct((n_nodes, n_out), jnp.float32),
        interpret=_INTERPRET,
    )(acc2d, w2d, bias)


def kernel(features, receivers, relative_positions, window_support, a, kernel, bias):
    kh, kw_half, in_ch, out_ch = kernel.shape
    n_nodes = features.shape[0]
    n_edges = receivers.shape[0]

    kernel_flipped = -jnp.flip(kernel, axis=(0, 1))
    kernel_full = jnp.concatenate([kernel, kernel_flipped], axis=1)
    w2d = kernel_full.reshape(N_TAPS * in_ch, out_ch)

    p = jnp.clip(relative_positions / window_support, -1.0, 1.0)
    gh = (p[:, 0] + 1.0) * (0.5 * (KH - 1))
    gw = (p[:, 1] + 1.0) * (0.5 * (KW - 1))

    edge_block = 1024
    n_pad = (-n_edges) % edge_block
    e_tot = n_edges + n_pad

    def pad_r(x, fill):
        x = jnp.concatenate([x, jnp.full((n_pad,), fill, x.dtype)])
        return x.reshape(e_tot // edge_block, edge_block // 128, 128)

    ghr = pad_r(gh, 0.0)
    gwr = pad_r(gw, 0.0)
    recvr = pad_r(receivers.astype(jnp.int32), 0)
    sndr = pad_r(a.astype(jnp.int32), 0)
    mskr = pad_r(jnp.ones((n_edges,), jnp.float32), 0.0)

    acc = _phase1(ghr, gwr, recvr, sndr, mskr, features, edge_block)
    acc2d = acc.reshape(n_nodes, N_TAPS * in_ch)
    block_n = 1000 if n_nodes % 1000 == 0 else n_nodes
    out = _phase2(acc2d, w2d, bias.reshape(1, out_ch), block_n)
    return out


# speedup vs baseline: 11.6721x; 11.6721x over previous
import functools

import jax
import jax.numpy as jnp
from jax import lax
from jax.experimental import pallas as pl
from jax.experimental.pallas import tpu as pltpu
from jax.experimental.pallas import tpu_sc as plsc

KH = 4
KW = 4
N_TAPS = KH * KW
_INTERPRET = False

_NS = 16
_NC = 2
_RN = 500
_CHUNK = 2000
_EB = 32
_DUMP = _RN * N_TAPS
_ACC_ROWS = 8192


def _sc_body(recv_hbm, snd_hbm, gh_hbm, gw_hbm, feat_hbm, acc_hbm,
             rbuf, sbuf, ghbuf, gwbuf, selr, sels, selgh, selgw,
             idx32, rix, wtb, cntb, xrows, srows, acc_sp, sem_io, sem_sc,
             sem_sc2, *, n_nodes, n_chunks, n_rounds):
    c = lax.axis_index("c")
    s = lax.axis_index("s")
    half = n_nodes // _NC
    z16f = jnp.zeros((16,), jnp.float32)
    z16i = jnp.zeros((16,), jnp.int32)

    def zsel(i, _):
        sels[pl.ds(i * 16, 16)] = z16i
        return 0
    lax.fori_loop(0, _CHUNK // 16, zsel, 0)
    for dd in range(2):
        for t in range(2):
            idx32[dd, pl.ds(t * 16, 16)] = z16i

    def round_body(r, _):
        def zrow(i, _):
            for t in range(8):
                srows[0, i, pl.ds(t * 16, 16)] = z16f
            return 0
        lax.fori_loop(0, 4 * _EB, zrow, 0)
        zb = s * 512
        for t in range(4):
            pltpu.sync_copy(srows.at[0], acc_sp.at[pl.ds(zb + t * 128, 128)])
        plsc.subcore_barrier()

        lo = c * half + r * _RN
        iota = lax.iota(jnp.int32, 16)
        one16 = jnp.ones((16,), jnp.int32)
        zero16 = jnp.zeros((16,), jnp.int32)

        def run_batches(nb, cnt):
            @pl.when(nb > 0)
            def _prologue():
                for t in range(2):
                    idx32[0, pl.ds(t * 16, 16)] = sels[pl.ds(t * 16, 16)]
                pltpu.async_copy(feat_hbm.at[idx32.at[0]], xrows.at[0],
                                 sem_io)

            def bloop(b, _):
                d = lax.rem(b, 2)
                nd = 1 - d
                @pl.when(jnp.logical_and(b >= 2, d == 0))
                def _drain0():
                    pltpu.make_async_copy(srows.at[0], acc_sp.at[rix.at[0]],
                                          sem_sc).wait()

                @pl.when(jnp.logical_and(b >= 2, d == 1))
                def _drain1():
                    pltpu.make_async_copy(srows.at[1], acc_sp.at[rix.at[1]],
                                          sem_sc2).wait()

                for grp in range(_EB // 16):
                    lb = b * _EB + grp * 16
                    ghv = selgh[pl.ds(lb, 16)]
                    gwv = selgw[pl.ds(lb, 16)]
                    h0v = jnp.clip(ghv.astype(jnp.int32), 0, KH - 1)
                    w0v = jnp.clip(gwv.astype(jnp.int32), 0, KW - 1)
                    fhv = ghv - h0v.astype(jnp.float32)
                    fwv = gwv - w0v.astype(jnp.float32)
                    h1v = jnp.minimum(h0v + 1, KH - 1)
                    w1v = jnp.minimum(w0v + 1, KW - 1)
                    rbv = selr[pl.ds(lb, 16)] * N_TAPS
                    validm = (lb + iota) < cnt
                    taps = [(h0v, w0v), (h0v, w1v), (h1v, w0v), (h1v, w1v)]
                    for j, (hv, wv) in enumerate(taps):
                        tap = rbv + hv * KW + wv
                        tap = jnp.where(validm, tap, _DUMP)
                        rix[d, pl.ds(j * _EB + grp * 16, 16)] = tap
                    wts = [(1.0 - fhv) * (1.0 - fwv), (1.0 - fhv) * fwv,
                           fhv * (1.0 - fwv), fhv * fwv]
                    for j, wv in enumerate(wts):
                        wtb[pl.ds(j * _EB + grp * 16, 16)] = wv

                pltpu.make_async_copy(feat_hbm.at[idx32.at[d]], xrows.at[d],
                                      sem_io).wait()

                @pl.when(b + 1 < nb)
                def _next_gather():
                    for t in range(2):
                        idx32[nd, pl.ds(t * 16, 16)] = (
                            sels[pl.ds((b + 1) * _EB + t * 16, 16)])
                    pltpu.async_copy(feat_hbm.at[idx32.at[nd]],
                                     xrows.at[nd], sem_io)

                nedge = jnp.minimum(_EB, cnt - b * _EB)

                def eloop(i, _):
                    w0s = wtb[pl.ds(0 * _EB + i, 16)][0]
                    w1s = wtb[pl.ds(1 * _EB + i, 16)][0]
                    w2s = wtb[pl.ds(2 * _EB + i, 16)][0]
                    w3s = wtb[pl.ds(3 * _EB + i, 16)][0]
                    for t in range(8):
                        xv = xrows[d, i, pl.ds(t * 16, 16)]
                        srows[d, 0 * _EB + i, pl.ds(t * 16, 16)] = xv * w0s
                        srows[d, 1 * _EB + i, pl.ds(t * 16, 16)] = xv * w1s
                        srows[d, 2 * _EB + i, pl.ds(t * 16, 16)] = xv * w2s
                        srows[d, 3 * _EB + i, pl.ds(t * 16, 16)] = xv * w3s
                    return 0

                lax.fori_loop(0, nedge, eloop, 0)

                @pl.when(d == 0)
                def _scat0():
                    pltpu.async_copy(srows.at[0], acc_sp.at[rix.at[0]],
                                     sem_sc, add=True)

                @pl.when(d == 1)
                def _scat1():
                    pltpu.async_copy(srows.at[1], acc_sp.at[rix.at[1]],
                                     sem_sc2, add=True)

                return 0

            lax.fori_loop(0, nb, bloop, 0)

            @pl.when(nb >= 1)
            def _drain_tail0():
                pltpu.make_async_copy(srows.at[0], acc_sp.at[rix.at[0]],
                                      sem_sc).wait()

            @pl.when(nb >= 2)
            def _drain_tail1():
                pltpu.make_async_copy(srows.at[1], acc_sp.at[rix.at[1]],
                                      sem_sc2).wait()

        def chunk_body(k, cnt_in):
            base_e = (s * n_chunks + k) * _CHUNK
            d1 = pltpu.async_copy(recv_hbm.at[pl.ds(base_e, _CHUNK)], rbuf, sem_io)
            d2 = pltpu.async_copy(snd_hbm.at[pl.ds(base_e, _CHUNK)], sbuf, sem_io)
            d3 = pltpu.async_copy(gh_hbm.at[pl.ds(base_e, _CHUNK)], ghbuf, sem_io)
            d4 = pltpu.async_copy(gw_hbm.at[pl.ds(base_e, _CHUNK)], gwbuf, sem_io)
            d1.wait()
            d2.wait()
            d3.wait()
            d4.wait()

            def vloop(v, cnt):
                rv = rbuf[pl.ds(v * 16, 16)] - lo
                m = jnp.logical_and(rv >= 0, rv < _RN)
                mi = jnp.where(m, one16, zero16)
                cs = mi
                for kk in (1, 2, 4, 8):
                    sh = jnp.take(cs, jnp.maximum(iota - kk, 0))
                    cs = cs + jnp.where(iota >= kk, sh, zero16)
                tgt = iota + 1
                pos = zero16
                for kk in (8, 4, 2, 1):
                    probe = jnp.take(cs, jnp.minimum(pos + (kk - 1), 15))
                    pos = pos + jnp.where(probe < tgt, kk * one16, zero16)
                inv = jnp.minimum(pos, 15)
                selr[pl.ds(cnt, 16)] = jnp.take(rv, inv)
                sels[pl.ds(cnt, 16)] = jnp.take(sbuf[pl.ds(v * 16, 16)], inv)
                selgh[pl.ds(cnt, 16)] = jnp.take(ghbuf[pl.ds(v * 16, 16)], inv)
                selgw[pl.ds(cnt, 16)] = jnp.take(gwbuf[pl.ds(v * 16, 16)], inv)
                cntb[pl.ds(0, 16)] = cs
                return cnt + cntb[pl.ds(0, 16)][15]

            cnt = lax.fori_loop(0, _CHUNK // 16, vloop, cnt_in)
            nb = lax.div(cnt, _EB)
            run_batches(nb, nb * _EB)
            rem = cnt - nb * _EB

            @pl.when(nb > 0)
            def _move_rem():
                for t in range(2):
                    selr[pl.ds(t * 16, 16)] = selr[pl.ds(nb * _EB + t * 16, 16)]
                    sels[pl.ds(t * 16, 16)] = sels[pl.ds(nb * _EB + t * 16, 16)]
                    selgh[pl.ds(t * 16, 16)] = (
                        selgh[pl.ds(nb * _EB + t * 16, 16)])
                    selgw[pl.ds(t * 16, 16)] = (
                        selgw[pl.ds(nb * _EB + t * 16, 16)])

            return rem

        cnt_left = lax.fori_loop(0, n_chunks, chunk_body, 0)
        run_batches(lax.div(cnt_left + (_EB - 1), _EB), cnt_left)
        plsc.subcore_barrier()

        @pl.when(s % 2 == 0)
        def _wb():
            blk = (s // 2) * 1000
            dst = (c * half + r * _RN) * N_TAPS + blk
            pltpu.sync_copy(acc_sp.at[pl.ds(blk, 1000)],
                            acc_hbm.at[pl.ds(dst, 1000)])

        plsc.subcore_barrier()
        return 0

    lax.fori_loop(0, n_rounds, round_body, 0)


def _sc_phase1(recv, snd, gh, gw, features):
    n_edges = recv.shape[0]
    n_nodes, n_ch = features.shape
    n_chunks = n_edges // (_NS * _CHUNK)
    assert n_edges == n_chunks * _NS * _CHUNK
    n_rounds = n_nodes // (_NC * _RN)
    assert n_nodes == n_rounds * _NC * _RN
    mesh = plsc.VectorSubcoreMesh(core_axis_name="c", subcore_axis_name="s")
    body = functools.partial(_sc_body, n_nodes=n_nodes, n_chunks=n_chunks,
                             n_rounds=n_rounds)
    return pl.kernel(
        body,
        out_type=jax.ShapeDtypeStruct((n_nodes * N_TAPS, n_ch), jnp.float32),
        mesh=mesh,
        scratch_types=[
            pltpu.VMEM((_CHUNK,), jnp.int32),
            pltpu.VMEM((_CHUNK,), jnp.int32),
            pltpu.VMEM((_CHUNK,), jnp.float32),
            pltpu.VMEM((_CHUNK,), jnp.float32),
            pltpu.VMEM((_CHUNK,), jnp.int32),
            pltpu.VMEM((_CHUNK,), jnp.int32),
            pltpu.VMEM((_CHUNK,), jnp.float32),
            pltpu.VMEM((_CHUNK,), jnp.float32),
            pltpu.VMEM((2, _EB), jnp.int32),
            pltpu.VMEM((2, 4 * _EB), jnp.int32),
            pltpu.VMEM((4 * _EB + 16,), jnp.float32),
            pltpu.VMEM((16,), jnp.int32),
            pltpu.VMEM((2, _EB, 128), jnp.float32),
            pltpu.VMEM((2, 4 * _EB, 128), jnp.float32),
            pltpu.VMEM_SHARED((_ACC_ROWS, 128), jnp.float32),
            pltpu.SemaphoreType.DMA,
            pltpu.SemaphoreType.DMA,
            pltpu.SemaphoreType.DMA,
        ],
    )(recv, snd, gh, gw, features)


def _matmul_body(acc_ref, w_ref, bias_ref, out_ref):
    out_ref[...] = (
        jnp.dot(acc_ref[...], w_ref[...], preferred_element_type=jnp.float32)
        + bias_ref[...])


def _phase2(acc2d, w2d, bias, block_n=1000):
    n_nodes = acc2d.shape[0]
    k = acc2d.shape[1]
    n_out = w2d.shape[1]
    grid = (n_nodes // block_n,)
    return pl.pallas_call(
        _matmul_body,
        grid=grid,
        in_specs=[pl.BlockSpec((block_n, k), lambda g: (g, 0)),
                  pl.BlockSpec((k, n_out), lambda g: (0, 0)),
                  pl.BlockSpec((1, n_out), lambda g: (0, 0))],
        out_specs=pl.BlockSpec((block_n, n_out), lambda g: (g, 0)),
        out_shape=jax.ShapeDtypeStruct((n_nodes, n_out), jnp.float32),
        interpret=_INTERPRET,
    )(acc2d, w2d, bias)


def kernel(features, receivers, relative_positions, window_support, a, kernel, bias):
    kh, kw_half, in_ch, out_ch = kernel.shape
    n_nodes = features.shape[0]

    kernel_flipped = -jnp.flip(kernel, axis=(0, 1))
    kernel_full = jnp.concatenate([kernel, kernel_flipped], axis=1)
    w2d = kernel_full.reshape(N_TAPS * in_ch, out_ch)

    p = jnp.clip(relative_positions / window_support, -1.0, 1.0)
    gh = (p[:, 0] + 1.0) * (0.5 * (KH - 1))
    gw = (p[:, 1] + 1.0) * (0.5 * (KW - 1))

    acc = _sc_phase1(receivers.astype(jnp.int32), a.astype(jnp.int32),
                     gh, gw, features)
    acc2d = acc.reshape(n_nodes, N_TAPS * in_ch)
    block_n = 1000 if n_nodes % 1000 == 0 else n_nodes
    return _phase2(acc2d, w2d, bias.reshape(1, out_ch), block_n)
